# baseline (speedup 1.0000x reference)
import functools

import jax
import jax.numpy as jnp
from jax import lax
from jax.experimental import pallas as pl
from jax.experimental.pallas import tpu as pltpu
from jax.experimental.pallas import tpu_sc as plsc

_D = 1000
_DP = 1024
_NC = 2
_NS = 16
_NW = _NC * _NS
_TP = 56


@functools.lru_cache(maxsize=None)
def _build(nb, t):
  b_per_w = nb // _NW
  assert b_per_w * _NW == nb

  mesh = plsc.VectorSubcoreMesh(
      core_axis_name="c", subcore_axis_name="s",
      num_cores=_NC, num_subcores=_NS)

  @functools.partial(
      pl.kernel,
      out_type=jax.ShapeDtypeStruct((nb, t, 8, 128), jnp.float32),
      mesh=mesh,
      scratch_types=[
          pltpu.VMEM((b_per_w * _TP,), jnp.int32),
          pltpu.VMEM((t, 8, 128), jnp.float32),
          pltpu.VMEM((t, 8, 128), jnp.float32),
          pltpu.SemaphoreType.DMA,
          pltpu.SemaphoreType.DMA,
          pltpu.SemaphoreType.DMA,
          pltpu.SemaphoreType.DMA,
      ],
  )
  def emb(idx_hbm, table_hbm, out_hbm,
          idx_v, buf0, buf1, gs0, gs1, ss0, ss1):
    wid = lax.axis_index("s") * _NC + lax.axis_index("c")
    b0 = wid * b_per_w
    pltpu.sync_copy(idx_hbm.at[pl.ds(b0 * _TP, b_per_w * _TP)], idx_v)

    bufs = ((buf0, gs0, ss0), (buf1, gs1, ss1))

    def gather(c, buf, gsem):
      return pltpu.make_async_copy(
          table_hbm.at[idx_v.at[pl.ds(c * _TP, t)]], buf, gsem)

    def scatter(c, buf, ssem):
      return pltpu.make_async_copy(buf, out_hbm.at[b0 + c], ssem)

    gather(0, buf0, gs0).start()

    @pl.loop(0, b_per_w, step=2)
    def _pair(j):
      for p in range(2):
        buf, gsem, ssem = bufs[p]
        c = j + p
        nxt = c + 1

        gather(c, buf, gsem).wait()
        scatter(c, buf, ssem).start()

        @pl.when(nxt < b_per_w)
        def _start_next():
          obuf, ogsem, ossem = bufs[1 - p]

          @pl.when(nxt >= 2)
          def _wait_prev_scatter():
            scatter(nxt - 2, obuf, ossem).wait()
          gather(nxt, obuf, ogsem).start()

    scatter(b_per_w - 2, buf0, ss0).wait()
    scatter(b_per_w - 1, buf1, ss1).wait()

  return emb


@jax.jit
def kernel(idx, table):
  b, t = idx.shape
  idx_p = jnp.pad(idx.astype(jnp.int32), ((0, 0), (0, _TP - t))).reshape(-1)
  table_t = jnp.pad(table, ((0, 0), (0, _DP - _D))).reshape(-1, 8, 128)
  ngroups = 4
  bg = b // ngroups
  parts = []
  for g in range(ngroups):
    idx_g = jax.lax.dynamic_slice_in_dim(idx_p, g * bg * _TP, bg * _TP)
    out4 = _build(bg, t)(idx_g, table_t)
    parts.append(out4.reshape(bg, t, _DP)[..., :_D])
  return jnp.concatenate(parts, axis=0)


# speedup vs baseline: 1.2281x; 1.2281x over previous
import functools

import jax
import jax.numpy as jnp
from jax import lax
from jax.experimental import pallas as pl
from jax.experimental.pallas import tpu as pltpu
from jax.experimental.pallas import tpu_sc as plsc

_D = 1000
_DP = 1024
_NC = 2
_NS = 16
_NW = _NC * _NS
_TP = 56


@functools.lru_cache(maxsize=None)
def _build(nb, t):
  b_per_w = nb // _NW
  assert b_per_w * _NW == nb

  mesh = plsc.VectorSubcoreMesh(
      core_axis_name="c", subcore_axis_name="s",
      num_cores=_NC, num_subcores=_NS)

  @functools.partial(
      pl.kernel,
      out_type=jax.ShapeDtypeStruct((nb, t, 8, 128), jnp.float32),
      mesh=mesh,
      scratch_types=[
          pltpu.VMEM((b_per_w * _TP,), jnp.int32),
          pltpu.VMEM((t, 8, 128), jnp.float32),
          pltpu.VMEM((t, 8, 128), jnp.float32),
          pltpu.SemaphoreType.DMA,
          pltpu.SemaphoreType.DMA,
          pltpu.SemaphoreType.DMA,
          pltpu.SemaphoreType.DMA,
      ],
  )
  def emb(idx_hbm, table_hbm, out_hbm,
          idx_v, buf0, buf1, gs0, gs1, ss0, ss1):
    wid = lax.axis_index("s") * _NC + lax.axis_index("c")
    b0 = wid * b_per_w
    pltpu.sync_copy(idx_hbm.at[pl.ds(b0 * _TP, b_per_w * _TP)], idx_v)

    bufs = ((buf0, gs0, ss0), (buf1, gs1, ss1))

    def gather(c, buf, gsem):
      return pltpu.make_async_copy(
          table_hbm.at[idx_v.at[pl.ds(c * _TP, t)]], buf, gsem)

    def scatter(c, buf, ssem):
      return pltpu.make_async_copy(buf, out_hbm.at[b0 + c], ssem)

    gather(0, buf0, gs0).start()

    @pl.loop(0, b_per_w, step=2)
    def _pair(j):
      for p in range(2):
        buf, gsem, ssem = bufs[p]
        c = j + p
        nxt = c + 1

        gather(c, buf, gsem).wait()
        scatter(c, buf, ssem).start()

        @pl.when(nxt < b_per_w)
        def _start_next():
          obuf, ogsem, ossem = bufs[1 - p]

          @pl.when(nxt >= 2)
          def _wait_prev_scatter():
            scatter(nxt - 2, obuf, ossem).wait()
          gather(nxt, obuf, ogsem).start()

    scatter(b_per_w - 2, buf0, ss0).wait()
    scatter(b_per_w - 1, buf1, ss1).wait()

  return emb


@jax.jit
def kernel(idx, table):
  b, t = idx.shape
  idx_p = jnp.pad(idx.astype(jnp.int32), ((0, 0), (0, _TP - t))).reshape(-1)
  table_t = jnp.pad(table, ((0, 0), (0, _DP - _D))).reshape(-1, 8, 128)
  out4 = _build(b, t)(idx_p, table_t)
  return out4.reshape(b, t, _DP)[..., :_D]
